# baseline (speedup 1.0000x reference)
import functools
import math

import jax
import jax.numpy as jnp
from jax import lax
from jax.experimental import pallas as pl
from jax.experimental.pallas import tpu as pltpu
from jax.experimental.pallas import tpu_sc as plsc

S = 200
B = 1024
D = 64
N = S * B
VOCAB = 1000000
MAX_LEN = 1000

NC = 2
NS = 16
NW = NC * NS
CHUNK = 512
NCHUNK = N // CHUNK
IDXW = 128
QPC = CHUNK // IDXW
FULL_T = NCHUNK // NW


def _make_pe():
    position = jnp.arange(MAX_LEN, dtype=jnp.float32)[:, None]
    div_term = jnp.exp(
        jnp.arange(0, D, 2, dtype=jnp.float32) * -(math.log(10000.0) / D))
    ang = position * div_term
    pe = jnp.zeros((MAX_LEN, D), dtype=jnp.float32)
    pe = pe.at[:, 0::2].set(jnp.sin(ang))
    pe = pe.at[:, 1::2].set(jnp.cos(ang))
    return pe[:S]


_mesh = plsc.VectorSubcoreMesh(core_axis_name="c", subcore_axis_name="s")


@functools.partial(
    pl.kernel,
    mesh=_mesh,
    compiler_params=pltpu.CompilerParams(use_tc_tiling_on_sc=False),
    out_type=jax.ShapeDtypeStruct((N, D), jnp.float32),
    scratch_types=[
        pltpu.VMEM((QPC, IDXW), jnp.int32),
        pltpu.VMEM((CHUNK, D), jnp.float32),
        pltpu.VMEM((S, D), jnp.float32),
        pltpu.SemaphoreType.DMA,
    ],
)
def _emb(idx_hbm, table_hbm, pe_hbm, out_hbm, idx_v, rows_v, pe_v, sem):
    wid = lax.axis_index("s") * NC + lax.axis_index("c")
    pltpu.sync_copy(pe_hbm, pe_v)

    def do_chunk(c):
        base = c * CHUNK
        s = c // (B // CHUNK)
        pltpu.sync_copy(idx_hbm.at[pl.ds(c * QPC, QPC)], idx_v)
        cps = [
            pltpu.async_copy(
                table_hbm.at[idx_v.at[q]],
                rows_v.at[pl.ds(q * IDXW, IDXW)],
                sem,
            )
            for q in range(QPC)
        ]
        for cp in cps:
            cp.wait()
        pe_regs = [pe_v[s, pl.ds(16 * j, 16)] for j in range(D // 16)]

        def body(r, carry):
            for j in range(D // 16):
                rows_v[r, pl.ds(16 * j, 16)] = (
                    rows_v[r, pl.ds(16 * j, 16)] + pe_regs[j])
            return carry

        lax.fori_loop(0, CHUNK, body, 0)
        pltpu.sync_copy(rows_v, out_hbm.at[pl.ds(base, CHUNK)])

    for t in range(FULL_T):
        do_chunk(wid + NW * t)

    c_last = wid + NW * FULL_T

    @pl.when(c_last < NCHUNK)
    def _():
        do_chunk(c_last)


def kernel(sequence, W):
    idx = sequence.reshape(NCHUNK * QPC, IDXW)
    pe = _make_pe()
    out = _emb(idx, W, pe)
    return out.reshape(S, B, D)


# speedup vs baseline: 1.0497x; 1.0497x over previous
import functools
import math

import jax
import jax.numpy as jnp
from jax import lax
from jax.experimental import pallas as pl
from jax.experimental.pallas import tpu as pltpu
from jax.experimental.pallas import tpu_sc as plsc

S = 200
B = 1024
D = 64
N = S * B
MAX_LEN = 1000

NC = 2
NS = 16
NW = NC * NS
CHUNK = 640
NCHUNK = N // CHUNK
IDXW = 128
QPC = CHUNK // IDXW
T = NCHUNK // NW
RSTEP = 4
NJ = D // 16


def _make_pe():
    position = jnp.arange(MAX_LEN, dtype=jnp.float32)[:, None]
    div_term = jnp.exp(
        jnp.arange(0, D, 2, dtype=jnp.float32) * -(math.log(10000.0) / D))
    ang = position * div_term
    pe = jnp.zeros((MAX_LEN, D), dtype=jnp.float32)
    pe = pe.at[:, 0::2].set(jnp.sin(ang))
    pe = pe.at[:, 1::2].set(jnp.cos(ang))
    return pe[:S]


_mesh = plsc.VectorSubcoreMesh(core_axis_name="c", subcore_axis_name="s")


@functools.partial(
    pl.kernel,
    mesh=_mesh,
    compiler_params=pltpu.CompilerParams(use_tc_tiling_on_sc=False),
    out_type=jax.ShapeDtypeStruct((N, D), jnp.float32),
    scratch_types=[
        pltpu.VMEM((2, QPC, IDXW), jnp.int32),
        pltpu.VMEM((2, CHUNK, D), jnp.float32),
        pltpu.VMEM((S, D), jnp.float32),
        pltpu.SemaphoreType.DMA,
        pltpu.SemaphoreType.DMA,
        pltpu.SemaphoreType.DMA,
    ],
)
def _emb(idx_hbm, table_hbm, pe_hbm, out_hbm, idx_v, rows_v, pe_v,
         sem_i, sem_g, sem_o):
    wid = lax.axis_index("s") * NC + lax.axis_index("c")
    pltpu.sync_copy(pe_hbm, pe_v)

    def chunk_id(t):
        return wid + NW * t

    def fire_idx(t):
        return pltpu.async_copy(
            idx_hbm.at[pl.ds(chunk_id(t) * QPC, QPC)],
            idx_v.at[t % 2], sem_i)

    def fire_gathers(t):
        p = t % 2
        return [
            pltpu.async_copy(
                table_hbm.at[idx_v.at[p, q]],
                rows_v.at[p, pl.ds(q * IDXW, IDXW)], sem_g)
            for q in range(QPC)
        ]

    def add_pe(t):
        p = t % 2
        c = chunk_id(t)
        base = c * CHUNK
        off = base % B
        s_lo = base // B
        r0 = jnp.minimum(B - off, CHUNK)
        s_hi = jnp.minimum(s_lo + 1, S - 1)
        pe_lo = [pe_v[s_lo, pl.ds(16 * j, 16)] for j in range(NJ)]
        pe_hi = [pe_v[s_hi, pl.ds(16 * j, 16)] for j in range(NJ)]

        @plsc.parallel_loop(0, r0, step=RSTEP, unroll=2)
        def _(r):
            for rr in range(RSTEP):
                for j in range(NJ):
                    rows_v[p, r + rr, pl.ds(16 * j, 16)] = (
                        rows_v[p, r + rr, pl.ds(16 * j, 16)] + pe_lo[j])

        @plsc.parallel_loop(r0, CHUNK, step=RSTEP, unroll=2)
        def _(r):
            for rr in range(RSTEP):
                for j in range(NJ):
                    rows_v[p, r + rr, pl.ds(16 * j, 16)] = (
                        rows_v[p, r + rr, pl.ds(16 * j, 16)] + pe_hi[j])

    def fire_out(t):
        return pltpu.async_copy(
            rows_v.at[t % 2],
            out_hbm.at[pl.ds(chunk_id(t) * CHUNK, CHUNK)], sem_o)

    fire_idx(0).wait()
    g_cps = {0: fire_gathers(0)}
    i_cps = {1: fire_idx(1)} if T > 1 else {}
    o_cps = {}

    for t in range(T):
        for cp in g_cps.pop(t):
            cp.wait()
        if t - 1 in o_cps:
            o_cps.pop(t - 1).wait()
        if t + 1 < T:
            i_cps.pop(t + 1).wait()
            g_cps[t + 1] = fire_gathers(t + 1)
        if t + 2 < T:
            i_cps[t + 2] = fire_idx(t + 2)
        add_pe(t)
        o_cps[t] = fire_out(t)

    o_cps.pop(T - 1).wait()


def kernel(sequence, W):
    idx = sequence.reshape(N // IDXW, IDXW)
    pe = _make_pe()
    out = _emb(idx, W, pe)
    return out.reshape(S, B, D)
